# baseline (speedup 1.0000x reference)
import functools

import jax
import jax.numpy as jnp
from jax import lax
from jax.experimental import pallas as pl
from jax.experimental.pallas import tpu as pltpu
from jax.experimental.pallas import tpu_sc as plsc

V = 1000000
D = 64
R = 8
SCALING = 2.0

NC, NS = 2, 16
NW = NC * NS
BB, LL = 1024, 200
TOK = BB * LL
PW = TOK // NW
TG = 256
NG = PW // TG
NSLOT = 4
MB = TG // 16


def _sc_body(x_ref, w_ref, a_ref, b_ref, out_ref,
             idx_v, wrows_v, arows_v, bt_v, wsem, asem, osem):
    cid = lax.axis_index("c")
    sid = lax.axis_index("s")
    wid = sid * NC + cid
    tok0 = wid * PW

    pltpu.sync_copy(x_ref.at[pl.ds(tok0, PW)], idx_v)
    pltpu.sync_copy(b_ref, bt_v)

    bts = [[bt_v[r, pl.ds(k * 16, 16)] for k in range(D // 16)]
           for r in range(R)]
    rconsts = [jnp.full((16,), r, jnp.int32) for r in range(R)]

    def fire_group(g, slot):
        for m in range(MB):
            ivec = idx_v[pl.ds(g * TG + m * 16, 16)]
            pltpu.make_async_copy(
                w_ref.at[ivec],
                wrows_v.at[slot, pl.ds(m * 16, 16)], wsem).start()
            pltpu.make_async_copy(
                a_ref.at[ivec],
                arows_v.at[slot, pl.ds(m * 16, 16)], asem).start()

    def drain_group(slot):
        pltpu.make_async_copy(
            w_ref.at[pl.ds(0, TG)], wrows_v.at[slot], wsem).wait()
        pltpu.make_async_copy(
            a_ref.at[pl.ds(0, TG)], arows_v.at[slot], asem).wait()

    def outcopy(g, slot):
        return pltpu.make_async_copy(
            wrows_v.at[slot], out_ref.at[pl.ds(tok0 + g * TG, TG)], osem)

    def compute(slot):
        def tok_body(t, tc):
            accs = [wrows_v[slot, t, pl.ds(k * 16, 16)]
                    for k in range(D // 16)]
            tvec = jnp.full((16,), t, jnp.int32)
            for r in range(R):
                ar = plsc.load_gather(arows_v.at[slot], [tvec, rconsts[r]])
                for k in range(D // 16):
                    accs[k] = accs[k] + ar * bts[r][k]
            for k in range(D // 16):
                wrows_v[slot, t, pl.ds(k * 16, 16)] = accs[k]
            return tc
        lax.fori_loop(0, TG, tok_body, 0, unroll=2)

    for gp in range(NSLOT - 1):
        fire_group(gp, gp)

    def group_body(g, carry):
        slot = lax.rem(g, NSLOT)

        @pl.when(g + NSLOT - 1 < NG)
        def _fire_ahead():
            @pl.when(g >= 1)
            def _drain_prev_write():
                outcopy(g - 1, lax.rem(g - 1, NSLOT)).wait()
            fire_group(g + NSLOT - 1, lax.rem(g + NSLOT - 1, NSLOT))

        drain_group(slot)
        compute(slot)
        outcopy(g, slot).start()
        return carry

    lax.fori_loop(0, NG, group_body, 0)
    for g in range(NG - NSLOT, NG):
        outcopy(g, g % NSLOT).wait()


@functools.cache
def _sc_lora_embed():
    return functools.partial(
        pl.kernel,
        out_type=jax.ShapeDtypeStruct((TOK, D), jnp.float32),
        mesh=plsc.VectorSubcoreMesh(core_axis_name="c", subcore_axis_name="s"),
        compiler_params=pltpu.CompilerParams(
            use_tc_tiling_on_sc=False, needs_layout_passes=False),
        scratch_types=[
            pltpu.VMEM((PW,), jnp.int32),
            pltpu.VMEM((NSLOT, TG, D), jnp.float32),
            pltpu.VMEM((NSLOT, TG, R), jnp.float32),
            pltpu.VMEM((R, D), jnp.float32),
            pltpu.SemaphoreType.DMA,
            pltpu.SemaphoreType.DMA,
            pltpu.SemaphoreType.DMA,
        ],
    )(_sc_body)


@jax.jit
def kernel(x, weight, lora_A, lora_B):
    x1 = x.reshape(TOK)
    a_t = (SCALING * lora_A).T
    bt = lora_B.T
    out = _sc_lora_embed()(x1, weight, a_t, bt)
    return out.reshape(BB, LL, D)
